# baseline (speedup 1.0000x reference)
import functools

import jax
import jax.numpy as jnp
from jax import lax
from jax.experimental import pallas as pl
from jax.experimental.pallas import tpu as pltpu
from jax.experimental.pallas import tpu_sc as plsc

N_NODES = 10000
N_EDGES = 320000
D = 128

NC = 2
NS = 16
L = 16
NW = NC * NS
EPW = N_EDGES // NW
C = 80
M = EPW // C
G = C // L


def _prescale_body(emb_ref, w_ref, we_ref, eb_ref):
    e = emb_ref[...]
    we_ref[...] = (e * w_ref[...]).astype(jnp.bfloat16)
    eb_ref[...] = e.astype(jnp.bfloat16)


def _prescale(embeddings, W):
    return pl.pallas_call(
        _prescale_body,
        out_shape=(jax.ShapeDtypeStruct((N_NODES, D), jnp.bfloat16),
                   jax.ShapeDtypeStruct((N_NODES, D), jnp.bfloat16)),
    )(embeddings, W)


_mesh = plsc.VectorSubcoreMesh(core_axis_name="c", subcore_axis_name="s")


@functools.partial(
    pl.kernel,
    out_type=jax.ShapeDtypeStruct((NW, M, C), jnp.float32),
    mesh=_mesh,
    scratch_types=[
        pltpu.VMEM((M, C), jnp.int32),
        pltpu.VMEM((M, C), jnp.int32),
        pltpu.VMEM((M, C), jnp.float32),
        pltpu.VMEM((C, D), jnp.uint32),
        pltpu.VMEM((C, D), jnp.uint32),
        pltpu.VMEM((C, D), jnp.uint32),
        pltpu.VMEM((C, D), jnp.uint32),
        pltpu.VMEM((L,), jnp.float32),
        pltpu.VMEM((C, L), jnp.float32),
        pltpu.SemaphoreType.DMA,
        pltpu.SemaphoreType.DMA,
        pltpu.SemaphoreType.DMA,
        pltpu.SemaphoreType.DMA,
    ],
    compiler_params=pltpu.CompilerParams(
        needs_layout_passes=False, disable_bounds_checks=True),
)
def _sc_edge_kernel(we_hbm, emb_hbm, srcidx_hbm, dstidx_hbm, b_hbm, out_hbm,
                    sidx_v, didx_v, out_v, sbuf0, dbuf0, sbuf1, dbuf1, b_v,
                    psum_v, sem_s0, sem_d0, sem_s1, sem_d1):
    wid = lax.axis_index("s") * NC + lax.axis_index("c")
    pltpu.sync_copy(srcidx_hbm.at[wid], sidx_v)
    pltpu.sync_copy(dstidx_hbm.at[wid], didx_v)
    pltpu.sync_copy(b_hbm, b_v)
    bvec = b_v[...]
    lane = lax.iota(jnp.int32, L)

    def _gather(cc, sb, db, ss, sd):
        pltpu.async_copy(we_hbm.at[sidx_v.at[cc]], sb, ss)
        pltpu.async_copy(emb_hbm.at[didx_v.at[cc]], db, sd)

    NSEG = D // (2 * L)

    def _compute(cc, sb, db, ss, sd):
        pltpu.make_async_copy(we_hbm.at[sidx_v.at[cc]], sb, ss).wait()
        pltpu.make_async_copy(emb_hbm.at[didx_v.at[cc]], db, sd).wait()

        @pl.loop(0, G)
        def _dots(g):
            e0 = g * L
            for j in range(L):
                e = e0 + j
                prods = [
                    plsc.bitcast(sb[e, pl.ds(k * L, L)], jnp.bfloat16)
                    * plsc.bitcast(db[e, pl.ds(D // 2 + k * L, L)],
                                   jnp.bfloat16)
                    for k in range(NSEG)]
                t = (prods[0] + prods[1]) + (prods[2] + prods[3])
                lo, hi = plsc.unpack(t, format=plsc.PackFormat.INTERLEAVED)
                psum_v[e] = lo + hi

        @pl.loop(0, G)
        def _reduce(g):
            e0 = g * L
            rows = jnp.full((L,), e0, jnp.int32) + lane
            acc = [jnp.zeros((L,), jnp.float32) for _ in range(4)]
            for r in range(L):
                col = plsc.load_gather(psum_v, [rows, jnp.full((L,), r, jnp.int32)])
                acc[r % 4] = acc[r % 4] + col
            x = acc[0] + acc[1] + (acc[2] + acc[3]) + bvec
            p = 1.0 / (1.0 + jnp.exp(-x))
            out_v[cc, pl.ds(e0, L)] = p

    _gather(0, sbuf0, dbuf0, sem_s0, sem_d0)

    @pl.loop(0, M - 1, step=2)
    def _chunk(c):
        _gather(c + 1, sbuf1, dbuf1, sem_s1, sem_d1)
        _compute(c, sbuf0, dbuf0, sem_s0, sem_d0)
        _gather(c + 2, sbuf0, dbuf0, sem_s0, sem_d0)
        _compute(c + 1, sbuf1, dbuf1, sem_s1, sem_d1)

    _compute(M - 1, sbuf0, dbuf0, sem_s0, sem_d0)

    pltpu.sync_copy(out_v, out_hbm.at[wid])


def kernel(embeddings, edges, W, b):
    we16, eb16 = _prescale(embeddings, W)
    tab = jnp.concatenate(
        [jax.lax.bitcast_convert_type(we16.reshape(N_NODES, D // 2, 2),
                                      jnp.uint32),
         jax.lax.bitcast_convert_type(eb16.reshape(N_NODES, D // 2, 2),
                                      jnp.uint32)], axis=1)
    src = edges[0].reshape(NW, M, C)
    dst = edges[1].reshape(NW, M, C)
    b16 = jnp.full((L,), b[0], jnp.float32)
    out = _sc_edge_kernel(tab, tab, src, dst, b16)
    return out.reshape(N_EDGES)


# speedup vs baseline: 1.2092x; 1.2092x over previous
import functools

import jax
import jax.numpy as jnp
from jax import lax
from jax.experimental import pallas as pl
from jax.experimental.pallas import tpu as pltpu
from jax.experimental.pallas import tpu_sc as plsc

N_NODES = 10000
N_EDGES = 320000
D = 128

NC = 2
NS = 16
L = 16
NW = NC * NS
EPW = N_EDGES // NW
C = 80
M = EPW // C
G = C // L


def _prescale_body(emb_ref, w_ref, out_ref):
    out_ref[...] = emb_ref[...] * w_ref[...]


def _prescale(embeddings, W):
    return pl.pallas_call(
        _prescale_body,
        out_shape=jax.ShapeDtypeStruct((N_NODES, D), jnp.float32),
    )(embeddings, W)


_mesh = plsc.VectorSubcoreMesh(core_axis_name="c", subcore_axis_name="s")


@functools.partial(
    pl.kernel,
    out_type=jax.ShapeDtypeStruct((NW, M, C), jnp.float32),
    mesh=_mesh,
    scratch_types=[
        pltpu.VMEM((M, C), jnp.int32),
        pltpu.VMEM((M, C), jnp.int32),
        pltpu.VMEM((M, C), jnp.float32),
        pltpu.VMEM((C, D), jnp.float32),
        pltpu.VMEM((C, D), jnp.float32),
        pltpu.VMEM((C, D), jnp.float32),
        pltpu.VMEM((C, D), jnp.float32),
        pltpu.VMEM((L,), jnp.float32),
        pltpu.VMEM((C, L), jnp.float32),
        pltpu.SemaphoreType.DMA,
        pltpu.SemaphoreType.DMA,
        pltpu.SemaphoreType.DMA,
        pltpu.SemaphoreType.DMA,
    ],
    compiler_params=pltpu.CompilerParams(
        needs_layout_passes=False, disable_bounds_checks=True),
)
def _sc_edge_kernel(we_hbm, emb_hbm, srcidx_hbm, dstidx_hbm, b_hbm, out_hbm,
                    sidx_v, didx_v, out_v, sbuf0, dbuf0, sbuf1, dbuf1, b_v,
                    psum_v, sem_s0, sem_d0, sem_s1, sem_d1):
    wid = lax.axis_index("s") * NC + lax.axis_index("c")
    pltpu.sync_copy(srcidx_hbm.at[wid], sidx_v)
    pltpu.sync_copy(dstidx_hbm.at[wid], didx_v)
    pltpu.sync_copy(b_hbm, b_v)
    bvec = b_v[...]
    lane = lax.iota(jnp.int32, L)

    def _gather(cc, sb, db, ss, sd):
        pltpu.async_copy(we_hbm.at[sidx_v.at[cc]], sb, ss)
        pltpu.async_copy(emb_hbm.at[didx_v.at[cc]], db, sd)

    NSEG = D // L

    def _compute(cc, sb, db, ss, sd):
        pltpu.make_async_copy(we_hbm.at[sidx_v.at[cc]], sb, ss).wait()
        pltpu.make_async_copy(emb_hbm.at[didx_v.at[cc]], db, sd).wait()

        @pl.loop(0, G)
        def _dots(g):
            e0 = g * L
            for j in range(L):
                e = e0 + j
                prods = [sb[e, pl.ds(k * L, L)] * db[e, pl.ds(k * L, L)]
                         for k in range(NSEG)]
                while len(prods) > 1:
                    prods = [prods[i] + prods[i + 1]
                             for i in range(0, len(prods), 2)]
                psum_v[e] = prods[0]

        @pl.loop(0, G)
        def _reduce(g):
            e0 = g * L
            rows = jnp.full((L,), e0, jnp.int32) + lane
            acc = [jnp.zeros((L,), jnp.float32) for _ in range(4)]
            for r in range(L):
                col = plsc.load_gather(psum_v, [rows, jnp.full((L,), r, jnp.int32)])
                acc[r % 4] = acc[r % 4] + col
            x = acc[0] + acc[1] + (acc[2] + acc[3]) + bvec
            p = 1.0 / (1.0 + jnp.exp(-x))
            out_v[cc, pl.ds(e0, L)] = p

    _gather(0, sbuf0, dbuf0, sem_s0, sem_d0)

    @pl.loop(0, M - 1, step=2)
    def _chunk(c):
        _gather(c + 1, sbuf1, dbuf1, sem_s1, sem_d1)
        _compute(c, sbuf0, dbuf0, sem_s0, sem_d0)
        _gather(c + 2, sbuf0, dbuf0, sem_s0, sem_d0)
        _compute(c + 1, sbuf1, dbuf1, sem_s1, sem_d1)

    _compute(M - 1, sbuf0, dbuf0, sem_s0, sem_d0)

    pltpu.sync_copy(out_v, out_hbm.at[wid])


def kernel(embeddings, edges, W, b):
    we = _prescale(embeddings, W)
    src = edges[0].reshape(NW, M, C)
    dst = edges[1].reshape(NW, M, C)
    b16 = jnp.full((L,), b[0], jnp.float32)
    out = _sc_edge_kernel(we, embeddings, src, dst, b16)
    return out.reshape(N_EDGES)
